# Initial kernel scaffold; baseline (speedup 1.0000x reference)
import jax
import jax.numpy as jnp
from jax.experimental import pallas as pl


def kernel(x, edge_index, edge_attr, batch, W1, as1, ad1, We1, ae1, b1, g1, be1, W2, as2, ad2, We2, ae2, b2, g2, be2, Wm, bm):
    raise NotImplementedError("write your pallas kernel here")


# speedup vs baseline: 1.0641x; 1.0641x over previous
import jax
import jax.numpy as jnp
from jax.experimental import pallas as pl

_G = 16


def _prep_body(x_ref, w_ref, ws_ref, wd_ref, xl_ref, as_ref, ad_ref):
    x = x_ref[:]
    xl_ref[:] = jnp.dot(x, w_ref[:], preferred_element_type=jnp.float32)
    as_ref[:] = jnp.dot(x, ws_ref[:], preferred_element_type=jnp.float32)
    ad_ref[:] = jnp.dot(x, wd_ref[:], preferred_element_type=jnp.float32)


def _edge_body(asg_ref, adg_ref, ea_ref, m_ref, e_ref):
    al = asg_ref[:] + adg_ref[:]
    ea = ea_ref[:]
    m = m_ref[:]
    for k in range(ea.shape[1]):
        al = al + ea[:, k:k + 1] * m[k:k + 1, :]
    al = jnp.where(al >= 0, al, 0.2 * al)
    e_ref[:] = jnp.exp(al)


def _post_body(num_ref, b_ref, g_ref, be_ref, out_ref):
    h = num_ref[:] + b_ref[:]
    h = jnp.where(h >= 0, h, 0.01 * h)
    mu = jnp.mean(h, axis=-1, keepdims=True)
    var = jnp.mean((h - mu) ** 2, axis=-1, keepdims=True)
    out_ref[:] = (h - mu) / jnp.sqrt(var + 1e-5) * g_ref[:] + be_ref[:]


def _final_body(h_ref, wm_ref, bm_ref, batch_ref, hout_ref, pooled_ref):
    h = jnp.dot(h_ref[:], wm_ref[:], preferred_element_type=jnp.float32) + bm_ref[:]
    h = jnp.where(h >= 0, h, 0.01 * h)
    hout_ref[:] = h
    b = batch_ref[:]
    n = b.shape[0]
    gi = jax.lax.broadcasted_iota(jnp.int32, (n, _G), 1)
    onehot = (b == gi).astype(jnp.float32)
    pooled = jax.lax.dot_general(onehot, h, (((0,), (0,)), ((), ())),
                                 preferred_element_type=jnp.float32)
    cnt = jax.lax.dot_general(onehot, jnp.ones((n, 1), jnp.float32),
                              (((0,), (0,)), ((), ())),
                              preferred_element_type=jnp.float32)
    pooled_ref[:] = pooled / jnp.maximum(cnt, 1.0)


def _prep(x, W, Ws, Wd):
    n = x.shape[0]
    hc = W.shape[1]
    h = Ws.shape[1]
    return pl.pallas_call(
        _prep_body,
        out_shape=[
            jax.ShapeDtypeStruct((n, hc), jnp.float32),
            jax.ShapeDtypeStruct((n, h), jnp.float32),
            jax.ShapeDtypeStruct((n, h), jnp.float32),
        ],
    )(x, W, Ws, Wd)


def _edge(asg, adg, ea, M):
    ef, h = asg.shape
    ed = ea.shape[1]
    blk = 8192
    grid = pl.cdiv(ef, blk)
    return pl.pallas_call(
        _edge_body,
        grid=(grid,),
        in_specs=[
            pl.BlockSpec((blk, h), lambda i: (i, 0)),
            pl.BlockSpec((blk, h), lambda i: (i, 0)),
            pl.BlockSpec((blk, ed), lambda i: (i, 0)),
            pl.BlockSpec((ed, h), lambda i: (0, 0)),
        ],
        out_specs=pl.BlockSpec((blk, h), lambda i: (i, 0)),
        out_shape=jax.ShapeDtypeStruct((ef, h), jnp.float32),
    )(asg, adg, ea, M)


def _post(num, b, g, be):
    return pl.pallas_call(
        _post_body,
        out_shape=jax.ShapeDtypeStruct(num.shape, jnp.float32),
    )(num, b.reshape(1, -1), g.reshape(1, -1), be.reshape(1, -1))


def _final(h, Wm, bm, batch):
    n, hc = h.shape
    return pl.pallas_call(
        _final_body,
        out_shape=[
            jax.ShapeDtypeStruct((n, hc), jnp.float32),
            jax.ShapeDtypeStruct((_G, hc), jnp.float32),
        ],
    )(h, Wm, bm.reshape(1, -1), batch.reshape(-1, 1))


def _gat_layer(x, src_f, dst_f, ea, W, att_s, att_d, We, att_e, bias, g, be):
    n = x.shape[0]
    d = x.shape[1]
    H, C = att_s.shape
    hc = W.shape[1]
    Wr = W.reshape(d, H, C)
    Ws = jnp.einsum('dhc,hc->dh', Wr, att_s)
    Wd = jnp.einsum('dhc,hc->dh', Wr, att_d)
    xl, a_s, a_d = _prep(x, W, Ws, Wd)
    M = jnp.einsum('khc,hc->kh', We.reshape(We.shape[0], H, C), att_e)
    e = _edge(a_s[src_f], a_d[dst_f], ea, M)
    s = jax.ops.segment_sum(e, dst_f, num_segments=n)
    att = e / (s[dst_f] + 1e-16)
    xl3 = xl.reshape(n, H, C)
    num = jax.ops.segment_sum(xl3[src_f] * att[:, :, None], dst_f,
                              num_segments=n).reshape(n, hc)
    h = _post(num, bias, g, be)
    return h, att


def kernel(x, edge_index, edge_attr, batch, W1, as1, ad1, We1, ae1, b1, g1, be1,
           W2, as2, ad2, We2, ae2, b2, g2, be2, Wm, bm):
    n = x.shape[0]
    src = edge_index[0]
    dst = edge_index[1]
    ones = jnp.ones((src.shape[0],), dtype=x.dtype)
    deg = jax.ops.segment_sum(ones, dst, num_segments=n)
    loop_attr = jax.ops.segment_sum(edge_attr, dst, num_segments=n) \
        / jnp.maximum(deg, 1.0)[:, None]
    loop = jnp.arange(n, dtype=src.dtype)
    src_f = jnp.concatenate([src, loop])
    dst_f = jnp.concatenate([dst, loop])
    ea = jnp.concatenate([edge_attr, loop_attr], axis=0)

    h, att1 = _gat_layer(x, src_f, dst_f, ea, W1, as1, ad1, We1, ae1, b1, g1, be1)
    h, att2 = _gat_layer(h, src_f, dst_f, ea, W2, as2, ad2, We2, ae2, b2, g2, be2)
    hout, pooled = _final(h, Wm, bm, batch)
    return hout, pooled, att1, att2
